# baseline (device time: 87169 ns/iter reference)
import jax
import jax.numpy as jnp
from jax import lax
from jax.experimental import pallas as pl
from jax.experimental.pallas import tpu as pltpu

N_DEV = 4


def kernel(x, w_mat):
    m_global, k_per = x.shape
    _, n = w_mat.shape
    m_per = m_global // N_DEV

    def body(x_ref, w_ref, out_ref, acc_ref, comm_ref, send_sems, recv_sems):
        my = lax.axis_index("i")
        left = lax.rem(my - 1 + N_DEV, N_DEV)
        right = lax.rem(my + 1, N_DEV)

        barrier_sem = pltpu.get_barrier_semaphore()
        for nbr in (left, right):
            pl.semaphore_signal(
                barrier_sem, inc=1,
                device_id=(nbr,), device_id_type=pl.DeviceIdType.MESH,
            )
        pl.semaphore_wait(barrier_sem, 2)

        w_bf = w_ref[:, :].astype(jnp.bfloat16)
        for s in range(N_DEV):
            chunk = lax.rem(my - 1 - s + 2 * N_DEV, N_DEV)
            xa = x_ref[pl.ds(chunk * m_per, m_per), :].astype(jnp.bfloat16)
            acc_ref[s, :, :] = jnp.dot(
                xa, w_bf, preferred_element_type=jnp.float32
            ).astype(jnp.bfloat16)

        for s in range(N_DEV - 1):
            rdma = pltpu.make_async_remote_copy(
                src_ref=acc_ref.at[s],
                dst_ref=comm_ref.at[s],
                send_sem=send_sems.at[s],
                recv_sem=recv_sems.at[s],
                device_id=(right,),
                device_id_type=pl.DeviceIdType.MESH,
            )
            rdma.start()
            rdma.wait()
            if s < N_DEV - 2:
                acc_ref[s + 1, :, :] = (
                    comm_ref[s].astype(jnp.float32)
                    + acc_ref[s + 1].astype(jnp.float32)
                ).astype(jnp.bfloat16)
            else:
                out_ref[:, :] = jnp.maximum(
                    comm_ref[s].astype(jnp.float32)
                    + acc_ref[s + 1].astype(jnp.float32),
                    0.0,
                )

    return pl.pallas_call(
        body,
        out_shape=jax.ShapeDtypeStruct((m_per, n), jnp.float32),
        in_specs=[
            pl.BlockSpec(memory_space=pltpu.VMEM),
            pl.BlockSpec(memory_space=pltpu.VMEM),
        ],
        out_specs=pl.BlockSpec(memory_space=pltpu.VMEM),
        scratch_shapes=[
            pltpu.VMEM((N_DEV, m_per, n), jnp.bfloat16),
            pltpu.VMEM((N_DEV - 1, m_per, n), jnp.bfloat16),
            pltpu.SemaphoreType.DMA((N_DEV - 1,)),
            pltpu.SemaphoreType.DMA((N_DEV - 1,)),
        ],
        compiler_params=pltpu.CompilerParams(collective_id=0),
    )(x, w_mat)


# device time: 51870 ns/iter; 1.6805x vs baseline; 1.6805x over previous
import jax
import jax.numpy as jnp
from jax import lax
from jax.experimental import pallas as pl
from jax.experimental.pallas import tpu as pltpu

N_DEV = 4


def kernel(x, w_mat):
    m_global, k_per = x.shape
    _, n = w_mat.shape
    m_per = m_global // N_DEV
    nh = n // 2

    def body(x_ref, w_ref, out_ref,
             acc_r, acc_l, comm_r, comm_l,
             send_r, recv_r, send_l, recv_l):
        my = lax.axis_index("i")
        left = lax.rem(my - 1 + N_DEV, N_DEV)
        right = lax.rem(my + 1, N_DEV)

        barrier_sem = pltpu.get_barrier_semaphore()
        for nbr in (left, right):
            pl.semaphore_signal(
                barrier_sem, inc=1,
                device_id=(nbr,), device_id_type=pl.DeviceIdType.MESH,
            )
        pl.semaphore_wait(barrier_sem, 2)

        w_bf = w_ref[:, :].astype(jnp.bfloat16)

        def compute_chunk(offset, r_slot, l_slot):
            chunk = lax.rem(my + offset + N_DEV, N_DEV)
            xa = x_ref[pl.ds(chunk * m_per, m_per), :].astype(jnp.bfloat16)
            p = jnp.dot(xa, w_bf, preferred_element_type=jnp.float32)
            acc_r[r_slot, :, :] = p[:, :nh].astype(jnp.bfloat16)
            acc_l[l_slot, :, :] = p[:, nh:].astype(jnp.bfloat16)

        compute_chunk(-1, 0, 2)
        compute_chunk(+1, 2, 0)

        def make_step(s):
            rdma_r = pltpu.make_async_remote_copy(
                src_ref=acc_r.at[s], dst_ref=comm_r.at[s],
                send_sem=send_r.at[s], recv_sem=recv_r.at[s],
                device_id=(right,), device_id_type=pl.DeviceIdType.MESH,
            )
            rdma_l = pltpu.make_async_remote_copy(
                src_ref=acc_l.at[s], dst_ref=comm_l.at[s],
                send_sem=send_l.at[s], recv_sem=recv_l.at[s],
                device_id=(left,), device_id_type=pl.DeviceIdType.MESH,
            )
            rdma_r.start()
            rdma_l.start()
            return rdma_r, rdma_l

        r0, l0 = make_step(0)
        compute_chunk(+2, 1, 1)
        compute_chunk(0, 3, 3)
        r0.wait()
        l0.wait()
        acc_r[1, :, :] = comm_r[0] + acc_r[1]
        acc_l[1, :, :] = comm_l[0] + acc_l[1]

        r1, l1 = make_step(1)
        r1.wait()
        l1.wait()
        acc_r[2, :, :] = comm_r[1] + acc_r[2]
        acc_l[2, :, :] = comm_l[1] + acc_l[2]

        r2, l2 = make_step(2)
        r2.wait()
        l2.wait()
        out_ref[:, :nh] = jnp.maximum(
            comm_r[2].astype(jnp.float32) + acc_r[3].astype(jnp.float32), 0.0)
        out_ref[:, nh:] = jnp.maximum(
            comm_l[2].astype(jnp.float32) + acc_l[3].astype(jnp.float32), 0.0)

    return pl.pallas_call(
        body,
        out_shape=jax.ShapeDtypeStruct((m_per, n), jnp.float32),
        in_specs=[
            pl.BlockSpec(memory_space=pltpu.VMEM),
            pl.BlockSpec(memory_space=pltpu.VMEM),
        ],
        out_specs=pl.BlockSpec(memory_space=pltpu.VMEM),
        scratch_shapes=[
            pltpu.VMEM((N_DEV, m_per, nh), jnp.bfloat16),
            pltpu.VMEM((N_DEV, m_per, nh), jnp.bfloat16),
            pltpu.VMEM((N_DEV - 1, m_per, nh), jnp.bfloat16),
            pltpu.VMEM((N_DEV - 1, m_per, nh), jnp.bfloat16),
            pltpu.SemaphoreType.DMA((N_DEV - 1,)),
            pltpu.SemaphoreType.DMA((N_DEV - 1,)),
            pltpu.SemaphoreType.DMA((N_DEV - 1,)),
            pltpu.SemaphoreType.DMA((N_DEV - 1,)),
        ],
        compiler_params=pltpu.CompilerParams(collective_id=0),
    )(x, w_mat)


# device time: 47926 ns/iter; 1.8188x vs baseline; 1.0823x over previous
import jax
import jax.numpy as jnp
from jax import lax
from jax.experimental import pallas as pl
from jax.experimental.pallas import tpu as pltpu

N_DEV = 4
N_LANE = 4


def kernel(x, w_mat):
    m_global, k_per = x.shape
    _, n = w_mat.shape
    m_per = m_global // N_DEV
    nl = n // N_LANE

    def body(x_ref, w_ref, out_ref,
             acc_r0, acc_r1, acc_l0, acc_l1,
             comm_r0, comm_r1, comm_l0, comm_l1,
             send_r0, recv_r0, send_r1, recv_r1,
             send_l0, recv_l0, send_l1, recv_l1):
        my = lax.axis_index("i")
        left = lax.rem(my - 1 + N_DEV, N_DEV)
        right = lax.rem(my + 1, N_DEV)

        barrier_sem = pltpu.get_barrier_semaphore()
        for nbr in (left, right):
            pl.semaphore_signal(
                barrier_sem, inc=1,
                device_id=(nbr,), device_id_type=pl.DeviceIdType.MESH,
            )
        pl.semaphore_wait(barrier_sem, 2)

        w_bf = w_ref[:, :].astype(jnp.bfloat16)

        rings = [
            (acc_r0, comm_r0, send_r0, recv_r0, right, 0 * nl),
            (acc_l0, comm_l0, send_l0, recv_l0, left, 2 * nl),
            (acc_r1, comm_r1, send_r1, recv_r1, right, 1 * nl),
            (acc_l1, comm_l1, send_l1, recv_l1, left, 3 * nl),
        ]

        def compute_chunk(offset, r_slot, l_slot):
            chunk = lax.rem(my + offset + N_DEV, N_DEV)
            xa = x_ref[pl.ds(chunk * m_per, m_per), :].astype(jnp.bfloat16)
            p = jnp.dot(xa, w_bf, preferred_element_type=jnp.float32)
            acc_r0[r_slot, :, :] = p[:, 0 * nl:1 * nl].astype(jnp.bfloat16)
            acc_r1[r_slot, :, :] = p[:, 1 * nl:2 * nl].astype(jnp.bfloat16)
            acc_l0[l_slot, :, :] = p[:, 2 * nl:3 * nl].astype(jnp.bfloat16)
            acc_l1[l_slot, :, :] = p[:, 3 * nl:4 * nl].astype(jnp.bfloat16)

        def start(s, ring):
            acc, comm, ssem, rsem, tgt, _ = ring
            rdma = pltpu.make_async_remote_copy(
                src_ref=acc.at[s], dst_ref=comm.at[s],
                send_sem=ssem.at[s], recv_sem=rsem.at[s],
                device_id=(tgt,), device_id_type=pl.DeviceIdType.MESH,
            )
            rdma.start()
            return rdma

        compute_chunk(-1, 0, 2)
        compute_chunk(+1, 2, 0)

        inflight = {id(r[0]): start(0, r) for r in rings}
        compute_chunk(+2, 1, 1)
        compute_chunk(0, 3, 3)

        for s in range(N_DEV - 1):
            for ring in rings:
                acc, comm, _, _, _, col = ring
                inflight[id(acc)].wait()
                if s < N_DEV - 2:
                    acc[s + 1, :, :] = comm[s] + acc[s + 1]
                    inflight[id(acc)] = start(s + 1, ring)
                else:
                    out_ref[:, col:col + nl] = jnp.maximum(
                        comm[s].astype(jnp.float32)
                        + acc[s + 1].astype(jnp.float32),
                        0.0,
                    )

    return pl.pallas_call(
        body,
        out_shape=jax.ShapeDtypeStruct((m_per, n), jnp.float32),
        in_specs=[
            pl.BlockSpec(memory_space=pltpu.VMEM),
            pl.BlockSpec(memory_space=pltpu.VMEM),
        ],
        out_specs=pl.BlockSpec(memory_space=pltpu.VMEM),
        scratch_shapes=(
            [pltpu.VMEM((N_DEV, m_per, nl), jnp.bfloat16)] * 4
            + [pltpu.VMEM((N_DEV - 1, m_per, nl), jnp.bfloat16)] * 4
            + [pltpu.SemaphoreType.DMA((N_DEV - 1,))] * 8
        ),
        compiler_params=pltpu.CompilerParams(collective_id=0),
    )(x, w_mat)


# device time: 42245 ns/iter; 2.0634x vs baseline; 1.1345x over previous
import jax
import jax.numpy as jnp
from jax import lax
from jax.experimental import pallas as pl
from jax.experimental.pallas import tpu as pltpu

N_DEV = 4
N_LANE = 4


def kernel(x, w_mat):
    m_global, k_per = x.shape
    _, n = w_mat.shape
    m_per = m_global // N_DEV
    nl = n // N_LANE

    def body(x_hbm, w_hbm, out_hbm, xv, wv, ov, wb,
             acc_r0, acc_r1, acc_l0, acc_l1,
             comm_r0, comm_r1, comm_l0, comm_l1,
             in_sems, out_sems,
             send_r0, recv_r0, send_r1, recv_r1,
             send_l0, recv_l0, send_l1, recv_l1):
        my = lax.axis_index("i")
        left = lax.rem(my - 1 + N_DEV, N_DEV)
        right = lax.rem(my + 1, N_DEV)

        x_cp = {}
        for j, off in enumerate((-1, +1, +2, 0)):
            chunk = lax.rem(my + off + N_DEV, N_DEV)
            rows = pl.ds(chunk * m_per, m_per)
            x_cp[off] = pltpu.make_async_copy(
                x_hbm.at[rows, :], xv.at[rows, :], in_sems.at[j])
        w_cp = {}
        for j, lane in enumerate((0, 2, 1, 3)):
            cols = pl.ds(lane * nl, nl)
            w_cp[lane] = pltpu.make_async_copy(
                w_hbm.at[:, cols], wv.at[:, cols], in_sems.at[4 + j])
        for cp in (x_cp[-1], w_cp[0], x_cp[+1], w_cp[2], w_cp[1], w_cp[3],
                   x_cp[+2], x_cp[0]):
            cp.start()

        barrier_sem = pltpu.get_barrier_semaphore()
        for nbr in (left, right):
            pl.semaphore_signal(
                barrier_sem, inc=1,
                device_id=(nbr,), device_id_type=pl.DeviceIdType.MESH,
            )
        pl.semaphore_wait(barrier_sem, 2)

        rings = [
            (acc_r0, comm_r0, send_r0, recv_r0, right, 0 * nl),
            (acc_l0, comm_l0, send_l0, recv_l0, left, 2 * nl),
            (acc_r1, comm_r1, send_r1, recv_r1, right, 1 * nl),
            (acc_l1, comm_l1, send_l1, recv_l1, left, 3 * nl),
        ]

        def stage_x(off):
            x_cp[off].wait()

        def stage_w(lane):
            cols = pl.ds(lane * nl, nl)
            w_cp[lane].wait()
            wb[:, cols] = wv[:, cols].astype(jnp.bfloat16)

        def gemm(offset, lo_lane, n_lanes, dsts):
            chunk = lax.rem(my + offset + N_DEV, N_DEV)
            xa = xv[pl.ds(chunk * m_per, m_per), :].astype(jnp.bfloat16)
            wl = wb[:, lo_lane * nl:(lo_lane + n_lanes) * nl]
            p = jnp.dot(xa, wl, preferred_element_type=jnp.float32)
            for j, (acc, slot) in enumerate(dsts):
                acc[slot, :, :] = p[:, j * nl:(j + 1) * nl].astype(jnp.bfloat16)

        def start(s, ring):
            acc, comm, ssem, rsem, tgt, _ = ring
            rdma = pltpu.make_async_remote_copy(
                src_ref=acc.at[s], dst_ref=comm.at[s],
                send_sem=ssem.at[s], recv_sem=rsem.at[s],
                device_id=(tgt,), device_id_type=pl.DeviceIdType.MESH,
            )
            rdma.start()
            return rdma

        inflight = {}

        stage_x(-1)
        stage_w(0)
        gemm(-1, 0, 1, [(acc_r0, 0)])
        inflight[id(acc_r0)] = start(0, rings[0])
        stage_x(+1)
        stage_w(2)
        gemm(+1, 2, 1, [(acc_l0, 0)])
        inflight[id(acc_l0)] = start(0, rings[1])

        stage_w(1)
        gemm(-1, 1, 1, [(acc_r1, 0)])
        inflight[id(acc_r1)] = start(0, rings[2])
        stage_w(3)
        gemm(+1, 3, 1, [(acc_l1, 0)])
        inflight[id(acc_l1)] = start(0, rings[3])

        stage_x(+2)
        gemm(+2, 0, 4, [(acc_r0, 1), (acc_r1, 1), (acc_l0, 1), (acc_l1, 1)])
        gemm(+1, 0, 2, [(acc_r0, 2), (acc_r1, 2)])
        gemm(-1, 2, 2, [(acc_l0, 2), (acc_l1, 2)])
        stage_x(0)
        gemm(0, 0, 4, [(acc_r0, 3), (acc_r1, 3), (acc_l0, 3), (acc_l1, 3)])

        out_cps = []
        for s in range(N_DEV - 1):
            for lane_idx, ring in enumerate(rings):
                acc, comm, _, _, _, col = ring
                inflight[id(acc)].wait()
                if s < N_DEV - 2:
                    acc[s + 1, :, :] = comm[s] + acc[s + 1]
                    inflight[id(acc)] = start(s + 1, ring)
                else:
                    cols = pl.ds(col, nl)
                    ov[:, cols] = jnp.maximum(
                        comm[s].astype(jnp.float32)
                        + acc[s + 1].astype(jnp.float32),
                        0.0,
                    ).astype(jnp.bfloat16)
                    cp = pltpu.make_async_copy(
                        ov.at[:, cols], out_hbm.at[:, cols],
                        out_sems.at[lane_idx])
                    cp.start()
                    out_cps.append(cp)
        for cp in out_cps:
            cp.wait()

    return pl.pallas_call(
        body,
        out_shape=jax.ShapeDtypeStruct((m_per, n), jnp.bfloat16),
        in_specs=[
            pl.BlockSpec(memory_space=pl.ANY),
            pl.BlockSpec(memory_space=pl.ANY),
        ],
        out_specs=pl.BlockSpec(memory_space=pltpu.MemorySpace.HBM),
        scratch_shapes=(
            [
                pltpu.VMEM((m_global, k_per), jnp.float32),
                pltpu.VMEM((k_per, n), jnp.float32),
                pltpu.VMEM((m_per, n), jnp.bfloat16),
                pltpu.VMEM((k_per, n), jnp.bfloat16),
            ]
            + [pltpu.VMEM((N_DEV, m_per, nl), jnp.bfloat16)] * 4
            + [pltpu.VMEM((N_DEV - 1, m_per, nl), jnp.bfloat16)] * 4
            + [
                pltpu.SemaphoreType.DMA((8,)),
                pltpu.SemaphoreType.DMA((N_LANE,)),
            ]
            + [pltpu.SemaphoreType.DMA((N_DEV - 1,))] * 8
        ),
        compiler_params=pltpu.CompilerParams(collective_id=0),
    )(
        pltpu.with_memory_space_constraint(x, pltpu.MemorySpace.HBM),
        pltpu.with_memory_space_constraint(w_mat, pltpu.MemorySpace.HBM),
    )
